# initial kernel scaffold (unmeasured)
import jax
import jax.numpy as jnp
from jax import lax
from jax.experimental import pallas as pl
from jax.experimental.pallas import tpu as pltpu

WORLD = 8
E_LOCAL = 8


def kernel(x, router_W, route_idx, expert_W):
    n_tok, d_model = x.shape
    e_loc, _, d_hid = expert_W.shape
    n_experts = router_W.shape[1]

    expert_W = expert_W.astype(jnp.bfloat16)

    def body(x_ref, rw_ref, idx_ref, ew_ref, out_ref,
             comm_ref, send_sems, recv_sems, credit_sem):
        my = lax.axis_index("i")
        left = lax.rem(my - 1 + WORLD, WORLD)
        right = lax.rem(my + 1, WORLD)

        barrier = pltpu.get_barrier_semaphore()
        for nbr in (left, right):
            pl.semaphore_signal(barrier, inc=1, device_id=(nbr,),
                                device_id_type=pl.DeviceIdType.MESH)
        pl.semaphore_wait(barrier, 2)

        xs = x_ref[:, :]
        scores = jnp.dot(xs, rw_ref[:, :], preferred_element_type=jnp.float32)
        smax = jnp.max(scores, axis=1, keepdims=True)
        ids = lax.broadcasted_iota(jnp.int32, scores.shape, 1)
        r0 = idx_ref[:, 0:1]
        r1 = idx_ref[:, 1:2]
        s0 = jnp.sum(jnp.where(ids == r0, scores, 0.0), axis=1, keepdims=True)
        s1 = jnp.sum(jnp.where(ids == r1, scores, 0.0), axis=1, keepdims=True)
        w0 = jnp.exp(s0 - smax)
        w1 = jnp.exp(s1 - smax)
        den = w0 + w1
        w0 = w0 / den
        w1 = w1 / den

        xb = xs.astype(jnp.bfloat16)
        out_ref[:, :] = jnp.zeros((n_tok, d_hid), jnp.float32)

        for h in range(WORLD):
            slot = h % 2
            src_ref = ew_ref if h == 0 else comm_ref.at[slot]
            src_pos = lax.rem(my - h + WORLD, WORLD)

            if h < WORLD - 1:
                if h >= 1:
                    pl.semaphore_wait(credit_sem, 1)
                rdma = pltpu.make_async_remote_copy(
                    src_ref=src_ref,
                    dst_ref=comm_ref.at[1 - slot],
                    send_sem=send_sems.at[h],
                    recv_sem=recv_sems.at[h],
                    device_id=(right,),
                    device_id_type=pl.DeviceIdType.MESH,
                )
                rdma.start()

            def expert_step(e, acc, src_ref=src_ref, src_pos=src_pos):
                eid = src_pos * E_LOCAL + e
                g = (jnp.where(r0 == eid, w0, 0.0)
                     + jnp.where(r1 == eid, w1, 0.0))
                w_e = src_ref[e] if isinstance(src_ref, type(ew_ref)) else src_ref[e]
                return acc + g * jnp.dot(xb, w_e,
                                         preferred_element_type=jnp.float32)

            acc = lax.fori_loop(
                0, E_LOCAL, expert_step,
                jnp.zeros((n_tok, d_hid), jnp.float32))
            out_ref[:, :] += acc

            if h <= WORLD - 3:
                pl.semaphore_signal(credit_sem, inc=1, device_id=(left,),
                                    device_id_type=pl.DeviceIdType.MESH)
            if h < WORLD - 1:
                rdma.wait()

    return pl.pallas_call(
        body,
        out_shape=jax.ShapeDtypeStruct((n_tok, d_hid), jnp.float32),
        in_specs=[pl.BlockSpec(memory_space=pltpu.VMEM)] * 4,
        out_specs=pl.BlockSpec(memory_space=pltpu.VMEM),
        scratch_shapes=[
            pltpu.VMEM((2, e_loc, d_model, d_hid), jnp.bfloat16),
            pltpu.SemaphoreType.DMA((WORLD - 1,)),
            pltpu.SemaphoreType.DMA((WORLD - 1,)),
            pltpu.SemaphoreType.REGULAR,
        ],
        compiler_params=pltpu.CompilerParams(collective_id=0),
    )(x, router_W, route_idx, expert_W)


# baseline (device time: 706533 ns/iter reference)
import jax
import jax.numpy as jnp
from jax import lax
from jax.experimental import pallas as pl
from jax.experimental.pallas import tpu as pltpu

WORLD = 8
E_LOCAL = 8


def kernel(x, router_W, route_idx, expert_W):
    n_tok, d_model = x.shape
    e_loc, _, d_hid = expert_W.shape

    scores = jnp.dot(x, router_W, preferred_element_type=jnp.float32)
    smax = jnp.max(scores, axis=1, keepdims=True)
    s01 = jnp.take_along_axis(scores, route_idx, axis=1)
    w01 = jnp.exp(s01 - smax)
    w01 = w01 / jnp.sum(w01, axis=1, keepdims=True)

    xb = x.astype(jnp.bfloat16)
    ew = expert_W.astype(jnp.bfloat16)

    def body(xb_ref, w01_ref, idx_ref, ew_ref, out_ref,
             comm_ref, send_sems, recv_sems, credit_sem):
        my = lax.axis_index("i")
        left = lax.rem(my - 1 + WORLD, WORLD)
        right = lax.rem(my + 1, WORLD)

        barrier = pltpu.get_barrier_semaphore()
        for nbr in (left, right):
            pl.semaphore_signal(barrier, inc=1, device_id=(nbr,),
                                device_id_type=pl.DeviceIdType.MESH)
        pl.semaphore_wait(barrier, 2)

        out_ref[:, :] = jnp.zeros((n_tok, d_hid), jnp.float32)

        def block_contrib(src_pos, w_of_e):
            def expert_step(e, carry):
                eid = src_pos * E_LOCAL + e
                g = (jnp.where(idx_ref[:, 0:1] == eid, w01_ref[:, 0:1], 0.0)
                     + jnp.where(idx_ref[:, 1:2] == eid, w01_ref[:, 1:2], 0.0))
                out_ref[:, :] += g * jnp.dot(
                    xb_ref[:, :], w_of_e(e),
                    preferred_element_type=jnp.float32)
                return carry
            lax.fori_loop(0, E_LOCAL, expert_step, 0)

        rdma0 = pltpu.make_async_remote_copy(
            src_ref=ew_ref,
            dst_ref=comm_ref.at[1],
            send_sem=send_sems.at[0],
            recv_sem=recv_sems.at[0],
            device_id=(right,),
            device_id_type=pl.DeviceIdType.MESH,
        )
        rdma0.start()
        block_contrib(my, lambda e: ew_ref[e])
        pl.semaphore_signal(credit_sem, inc=1, device_id=(left,),
                            device_id_type=pl.DeviceIdType.MESH)
        rdma0.wait()

        def hop_body(h, _):
            slot = lax.rem(h, 2)
            src_pos = lax.rem(my - h + WORLD, WORLD)
            rdma = pltpu.make_async_remote_copy(
                src_ref=comm_ref.at[slot],
                dst_ref=comm_ref.at[1 - slot],
                send_sem=send_sems.at[h],
                recv_sem=recv_sems.at[h],
                device_id=(right,),
                device_id_type=pl.DeviceIdType.MESH,
            )

            @pl.when(h < WORLD - 1)
            def _():
                pl.semaphore_wait(credit_sem, 1)
                rdma.start()

            block_contrib(src_pos, lambda e: comm_ref[slot, e])

            @pl.when(h <= WORLD - 3)
            def _():
                pl.semaphore_signal(credit_sem, inc=1, device_id=(left,),
                                    device_id_type=pl.DeviceIdType.MESH)

            @pl.when(h < WORLD - 1)
            def _():
                rdma.wait()

            return 0

        lax.fori_loop(1, WORLD, hop_body, 0)

    return pl.pallas_call(
        body,
        out_shape=jax.ShapeDtypeStruct((n_tok, d_hid), jnp.float32),
        in_specs=[pl.BlockSpec(memory_space=pltpu.VMEM)] * 4,
        out_specs=pl.BlockSpec(memory_space=pltpu.VMEM),
        scratch_shapes=[
            pltpu.VMEM((2, e_loc, d_model, d_hid), jnp.bfloat16),
            pltpu.SemaphoreType.DMA((WORLD,)),
            pltpu.SemaphoreType.DMA((WORLD,)),
            pltpu.SemaphoreType.REGULAR,
        ],
        compiler_params=pltpu.CompilerParams(
            collective_id=0,
            vmem_limit_bytes=60 * 1024 * 1024,
        ),
    )(xb, w01, route_idx, ew)


# device time: 391101 ns/iter; 1.8065x vs baseline; 1.8065x over previous
import jax
import jax.numpy as jnp
from jax import lax
from jax.experimental import pallas as pl
from jax.experimental.pallas import tpu as pltpu

WORLD = 8
E_LOCAL = 8
E_HALF = 4


def kernel(x, router_W, route_idx, expert_W):
    n_tok, d_model = x.shape
    e_loc, _, d_hid = expert_W.shape

    scores = jnp.dot(x, router_W, preferred_element_type=jnp.float32)
    smax = jnp.max(scores, axis=1, keepdims=True)
    s01 = jnp.take_along_axis(scores, route_idx, axis=1)
    w01 = jnp.exp(s01 - smax)
    w01 = w01 / jnp.sum(w01, axis=1, keepdims=True)

    xb = x.astype(jnp.bfloat16)
    ew = expert_W.astype(jnp.bfloat16)

    def body(xb_ref, w01_ref, idx_ref, ew_ref, out_ref,
             cw_ref, ccw_ref, cw_send, cw_recv, ccw_send, ccw_recv,
             credit_cw, credit_ccw):
        my = lax.axis_index("i")

        def to_dev(kk):
            return jnp.where(kk < 4, kk, 11 - kk)

        k = to_dev(my)
        right = to_dev(lax.rem(k + 1, WORLD))
        left = to_dev(lax.rem(k - 1 + WORLD, WORLD))

        barrier = pltpu.get_barrier_semaphore()
        for nbr in (left, right):
            pl.semaphore_signal(barrier, inc=1, device_id=(nbr,),
                                device_id_type=pl.DeviceIdType.MESH)
        pl.semaphore_wait(barrier, 2)

        out_ref[:, :] = jnp.zeros((n_tok, d_hid), jnp.float32)

        def half_contrib(src_pos, e_off, w_of_e):
            def expert_step(e, carry):
                eid = src_pos * E_LOCAL + e_off + e
                g = (jnp.where(idx_ref[:, 0:1] == eid, w01_ref[:, 0:1], 0.0)
                     + jnp.where(idx_ref[:, 1:2] == eid, w01_ref[:, 1:2], 0.0))
                out_ref[:, :] += g * jnp.dot(
                    xb_ref[:, :], w_of_e(e),
                    preferred_element_type=jnp.float32)
                return carry
            lax.fori_loop(0, E_HALF, expert_step, 0)

        rdma0_cw = pltpu.make_async_remote_copy(
            src_ref=ew_ref.at[0:E_HALF],
            dst_ref=cw_ref.at[1],
            send_sem=cw_send.at[0],
            recv_sem=cw_recv.at[0],
            device_id=(right,),
            device_id_type=pl.DeviceIdType.MESH,
        )
        rdma0_ccw = pltpu.make_async_remote_copy(
            src_ref=ew_ref.at[E_HALF:E_LOCAL],
            dst_ref=ccw_ref.at[1],
            send_sem=ccw_send.at[0],
            recv_sem=ccw_recv.at[0],
            device_id=(left,),
            device_id_type=pl.DeviceIdType.MESH,
        )
        rdma0_cw.start()
        rdma0_ccw.start()
        half_contrib(my, 0, lambda e: ew_ref[e])
        half_contrib(my, E_HALF, lambda e: ew_ref[E_HALF + e])
        pl.semaphore_signal(credit_cw, inc=1, device_id=(left,),
                            device_id_type=pl.DeviceIdType.MESH)
        pl.semaphore_signal(credit_ccw, inc=1, device_id=(right,),
                            device_id_type=pl.DeviceIdType.MESH)
        rdma0_cw.wait()
        rdma0_ccw.wait()

        def hop_body(h, _):
            slot = lax.rem(h, 2)
            src_cw = to_dev(lax.rem(k - h + WORLD, WORLD))
            src_ccw = to_dev(lax.rem(k + h, WORLD))
            rdma_cw = pltpu.make_async_remote_copy(
                src_ref=cw_ref.at[slot],
                dst_ref=cw_ref.at[1 - slot],
                send_sem=cw_send.at[h],
                recv_sem=cw_recv.at[h],
                device_id=(right,),
                device_id_type=pl.DeviceIdType.MESH,
            )
            rdma_ccw = pltpu.make_async_remote_copy(
                src_ref=ccw_ref.at[slot],
                dst_ref=ccw_ref.at[1 - slot],
                send_sem=ccw_send.at[h],
                recv_sem=ccw_recv.at[h],
                device_id=(left,),
                device_id_type=pl.DeviceIdType.MESH,
            )

            @pl.when(h < WORLD - 1)
            def _():
                pl.semaphore_wait(credit_cw, 1)
                rdma_cw.start()
                pl.semaphore_wait(credit_ccw, 1)
                rdma_ccw.start()

            half_contrib(src_cw, 0, lambda e: cw_ref[slot, e])
            half_contrib(src_ccw, E_HALF, lambda e: ccw_ref[slot, e])

            @pl.when(h <= WORLD - 3)
            def _():
                pl.semaphore_signal(credit_cw, inc=1, device_id=(left,),
                                    device_id_type=pl.DeviceIdType.MESH)
                pl.semaphore_signal(credit_ccw, inc=1, device_id=(right,),
                                    device_id_type=pl.DeviceIdType.MESH)

            @pl.when(h < WORLD - 1)
            def _():
                rdma_cw.wait()
                rdma_ccw.wait()

            return 0

        lax.fori_loop(1, WORLD, hop_body, 0)

    return pl.pallas_call(
        body,
        out_shape=jax.ShapeDtypeStruct((n_tok, d_hid), jnp.float32),
        in_specs=[pl.BlockSpec(memory_space=pltpu.VMEM)] * 4,
        out_specs=pl.BlockSpec(memory_space=pltpu.VMEM),
        scratch_shapes=[
            pltpu.VMEM((2, E_HALF, d_model, d_hid), jnp.bfloat16),
            pltpu.VMEM((2, E_HALF, d_model, d_hid), jnp.bfloat16),
            pltpu.SemaphoreType.DMA((WORLD,)),
            pltpu.SemaphoreType.DMA((WORLD,)),
            pltpu.SemaphoreType.DMA((WORLD,)),
            pltpu.SemaphoreType.DMA((WORLD,)),
            pltpu.SemaphoreType.REGULAR,
            pltpu.SemaphoreType.REGULAR,
        ],
        compiler_params=pltpu.CompilerParams(
            collective_id=0,
            vmem_limit_bytes=60 * 1024 * 1024,
        ),
    )(xb, w01, route_idx, ew)


# device time: 384332 ns/iter; 1.8383x vs baseline; 1.0176x over previous
import jax
import jax.numpy as jnp
from jax import lax
from jax.experimental import pallas as pl
from jax.experimental.pallas import tpu as pltpu

WORLD = 8
E_LOCAL = 8
E_HALF = 4


def kernel(x, router_W, route_idx, expert_W):
    n_tok, d_model = x.shape
    e_loc, _, d_hid = expert_W.shape

    scores = jnp.dot(x, router_W, preferred_element_type=jnp.float32)
    smax = jnp.max(scores, axis=1, keepdims=True)
    iota = jnp.arange(scores.shape[1], dtype=route_idx.dtype)[None, :]
    s0 = jnp.sum(jnp.where(route_idx[:, 0:1] == iota, scores, 0.0),
                 axis=1, keepdims=True)
    s1 = jnp.sum(jnp.where(route_idx[:, 1:2] == iota, scores, 0.0),
                 axis=1, keepdims=True)
    s01 = jnp.concatenate([s0, s1], axis=1)
    w01 = jnp.exp(s01 - smax)
    w01 = w01 / jnp.sum(w01, axis=1, keepdims=True)

    xb = x.astype(jnp.bfloat16)
    ew = expert_W.astype(jnp.bfloat16)

    def body(xb_ref, w01_ref, idx_ref, ew_ref, out_ref,
             cw_ref, ccw_ref, cw_send, cw_recv, ccw_send, ccw_recv,
             credit_cw, credit_ccw):
        my = lax.axis_index("i")

        def to_dev(kk):
            return jnp.where(kk < 4, kk, 11 - kk)

        k = to_dev(my)
        right = to_dev(lax.rem(k + 1, WORLD))
        left = to_dev(lax.rem(k - 1 + WORLD, WORLD))

        barrier = pltpu.get_barrier_semaphore()
        for nbr in (left, right):
            pl.semaphore_signal(barrier, inc=1, device_id=(nbr,),
                                device_id_type=pl.DeviceIdType.MESH)
        pl.semaphore_wait(barrier, 2)

        out_ref[:, :] = jnp.zeros((n_tok, d_hid), jnp.float32)

        def half_contrib(src_pos, e_off, w_of_e):
            def expert_step(e, carry):
                eid = src_pos * E_LOCAL + e_off + e
                g = (jnp.where(idx_ref[:, 0:1] == eid, w01_ref[:, 0:1], 0.0)
                     + jnp.where(idx_ref[:, 1:2] == eid, w01_ref[:, 1:2], 0.0))
                out_ref[:, :] += g * jnp.dot(
                    xb_ref[:, :], w_of_e(e),
                    preferred_element_type=jnp.float32)
                return carry
            lax.fori_loop(0, E_HALF, expert_step, 0)

        rdma0_cw = pltpu.make_async_remote_copy(
            src_ref=ew_ref.at[0:E_HALF],
            dst_ref=cw_ref.at[1],
            send_sem=cw_send.at[0],
            recv_sem=cw_recv.at[0],
            device_id=(right,),
            device_id_type=pl.DeviceIdType.MESH,
        )
        rdma0_ccw = pltpu.make_async_remote_copy(
            src_ref=ew_ref.at[E_HALF:E_LOCAL],
            dst_ref=ccw_ref.at[1],
            send_sem=ccw_send.at[0],
            recv_sem=ccw_recv.at[0],
            device_id=(left,),
            device_id_type=pl.DeviceIdType.MESH,
        )
        rdma0_cw.start()
        rdma0_ccw.start()
        half_contrib(my, 0, lambda e: ew_ref[e])
        half_contrib(my, E_HALF, lambda e: ew_ref[E_HALF + e])
        pl.semaphore_signal(credit_cw, inc=1, device_id=(left,),
                            device_id_type=pl.DeviceIdType.MESH)
        pl.semaphore_signal(credit_ccw, inc=1, device_id=(right,),
                            device_id_type=pl.DeviceIdType.MESH)
        rdma0_cw.wait()
        rdma0_ccw.wait()

        def hop_body(h, _):
            slot = lax.rem(h, 2)
            src_cw = to_dev(lax.rem(k - h + WORLD, WORLD))
            src_ccw = to_dev(lax.rem(k + h, WORLD))
            rdma_cw = pltpu.make_async_remote_copy(
                src_ref=cw_ref.at[slot],
                dst_ref=cw_ref.at[1 - slot],
                send_sem=cw_send.at[h],
                recv_sem=cw_recv.at[h],
                device_id=(right,),
                device_id_type=pl.DeviceIdType.MESH,
            )
            rdma_ccw = pltpu.make_async_remote_copy(
                src_ref=ccw_ref.at[slot],
                dst_ref=ccw_ref.at[1 - slot],
                send_sem=ccw_send.at[h],
                recv_sem=ccw_recv.at[h],
                device_id=(left,),
                device_id_type=pl.DeviceIdType.MESH,
            )

            @pl.when(h < WORLD - 1)
            def _():
                pl.semaphore_wait(credit_cw, 1)
                rdma_cw.start()
                pl.semaphore_wait(credit_ccw, 1)
                rdma_ccw.start()

            half_contrib(src_cw, 0, lambda e: cw_ref[slot, e])
            half_contrib(src_ccw, E_HALF, lambda e: ccw_ref[slot, e])

            @pl.when(h <= WORLD - 3)
            def _():
                pl.semaphore_signal(credit_cw, inc=1, device_id=(left,),
                                    device_id_type=pl.DeviceIdType.MESH)
                pl.semaphore_signal(credit_ccw, inc=1, device_id=(right,),
                                    device_id_type=pl.DeviceIdType.MESH)

            @pl.when(h < WORLD - 1)
            def _():
                rdma_cw.wait()
                rdma_ccw.wait()

            return 0

        lax.fori_loop(1, WORLD, hop_body, 0)

    return pl.pallas_call(
        body,
        out_shape=jax.ShapeDtypeStruct((n_tok, d_hid), jnp.float32),
        in_specs=[pl.BlockSpec(memory_space=pltpu.VMEM)] * 4,
        out_specs=pl.BlockSpec(memory_space=pltpu.VMEM),
        scratch_shapes=[
            pltpu.VMEM((2, E_HALF, d_model, d_hid), jnp.bfloat16),
            pltpu.VMEM((2, E_HALF, d_model, d_hid), jnp.bfloat16),
            pltpu.SemaphoreType.DMA((WORLD,)),
            pltpu.SemaphoreType.DMA((WORLD,)),
            pltpu.SemaphoreType.DMA((WORLD,)),
            pltpu.SemaphoreType.DMA((WORLD,)),
            pltpu.SemaphoreType.REGULAR,
            pltpu.SemaphoreType.REGULAR,
        ],
        compiler_params=pltpu.CompilerParams(
            collective_id=0,
            vmem_limit_bytes=60 * 1024 * 1024,
        ),
    )(xb, w01, route_idx, ew)
